# baseline (speedup 1.0000x reference)
import jax
import jax.numpy as jnp
from jax.experimental import pallas as pl
from jax.experimental.pallas import tpu as pltpu

_N, _L, _C = 16, 256, 1024
_K = 4
_G = 4
_W = _G * _L

_NT = (((1,), (1,)), ((), ()))


def _msm_kernel(z_all_ref, z_g_ref, out_ref, n_all_ref):
    c = pl.program_id(0)
    z_all = z_all_ref[...]
    z_g = z_g_ref[...]

    @pl.when(c == 0)
    def _():
        n_all_ref[...] = jnp.sum(z_all * z_all, axis=1, keepdims=True)

    g = jax.lax.dot_general(z_all.astype(jnp.bfloat16),
                            z_g.astype(jnp.bfloat16), _NT,
                            preferred_element_type=jnp.float32)
    h = n_all_ref[...] - 2.0 * g

    n_p = jax.lax.dot_general(jnp.ones((1, _C), jnp.float32), z_g * z_g, _NT,
                              preferred_element_type=jnp.float32)

    mins = [jnp.min(h[j * _L:(j + 1) * _L, :], axis=0, keepdims=True)
            for j in range(_N)]
    m = jnp.concatenate(mins, axis=0)

    row = jax.lax.broadcasted_iota(jnp.int32, (_N, _W), 0)
    img = c * _G + jax.lax.broadcasted_iota(jnp.int32, (_N, _W), 1) // _L
    inf = jnp.float32(jnp.inf)
    m = jnp.where(row == img, inf, m)

    acc = jnp.zeros((1, _W), jnp.float32)
    rem = jnp.full((1, _W), jnp.float32(_K))
    for _ in range(_K):
        v = jnp.min(m, axis=0, keepdims=True)
        cnt = jnp.sum((m == v).astype(jnp.float32), axis=0, keepdims=True)
        t = jnp.minimum(cnt, rem)
        d = jnp.sqrt(jnp.maximum(n_p + v, 1e-12))
        acc = acc + jnp.where(t > 0.0, t * d, 0.0)
        rem = rem - t
        m = jnp.where(m == v, inf, m)

    acc = acc / jnp.float32(_K)
    out_ref[...] = jnp.concatenate(
        [acc[:, k * _L:(k + 1) * _L] for k in range(_G)], axis=0
    ).reshape(_G, 1, _L)


def kernel(Z):
    N, L, C = Z.shape
    z_all = Z.reshape(N * L, C)
    out = pl.pallas_call(
        _msm_kernel,
        grid=(N // _G,),
        in_specs=[
            pl.BlockSpec((N * L, C), lambda c: (0, 0)),
            pl.BlockSpec((_W, C), lambda c: (c, 0)),
        ],
        out_specs=pl.BlockSpec((_G, 1, L), lambda c: (c, 0, 0)),
        out_shape=jax.ShapeDtypeStruct((N, 1, L), jnp.float32),
        scratch_shapes=[pltpu.VMEM((N * L, 1), jnp.float32)],
    )(z_all, z_all)
    return out.reshape(N, L)


# speedup vs baseline: 1.0800x; 1.0800x over previous
import jax
import jax.numpy as jnp
from jax.experimental import pallas as pl
from jax.experimental.pallas import tpu as pltpu

_N, _L, _C = 16, 256, 1024
_K = 4
_G = 4
_W = _G * _L

_NT = (((1,), (1,)), ((), ()))


def _msm_kernel(z_all_ref, z_g_ref, out_ref, n_all_ref, zb_ref):
    c = pl.program_id(0)
    z_g = z_g_ref[...]

    @pl.when(c == 0)
    def _():
        z_all = z_all_ref[...]
        n_all_ref[...] = jnp.sum(z_all * z_all, axis=1, keepdims=True)
        zb_ref[...] = z_all.astype(jnp.bfloat16)

    g = jax.lax.dot_general(zb_ref[...], z_g.astype(jnp.bfloat16), _NT,
                            preferred_element_type=jnp.float32)
    h = n_all_ref[...] - 2.0 * g

    n_p = jax.lax.dot_general(jnp.ones((1, _C), jnp.float32), z_g * z_g, _NT,
                              preferred_element_type=jnp.float32)

    mins = [jnp.min(h[j * _L:(j + 1) * _L, :], axis=0, keepdims=True)
            for j in range(_N)]
    m = jnp.concatenate(mins, axis=0)

    row = jax.lax.broadcasted_iota(jnp.int32, (_N, _W), 0)
    img = c * _G + jax.lax.broadcasted_iota(jnp.int32, (_N, _W), 1) // _L
    inf = jnp.float32(jnp.inf)
    m = jnp.where(row == img, inf, m)

    acc = jnp.zeros((1, _W), jnp.float32)
    rem = jnp.full((1, _W), jnp.float32(_K))
    for _ in range(_K):
        v = jnp.min(m, axis=0, keepdims=True)
        cnt = jnp.sum((m == v).astype(jnp.float32), axis=0, keepdims=True)
        t = jnp.minimum(cnt, rem)
        d = jnp.sqrt(jnp.maximum(n_p + v, 1e-12))
        acc = acc + jnp.where(t > 0.0, t * d, 0.0)
        rem = rem - t
        m = jnp.where(m == v, inf, m)

    acc = acc / jnp.float32(_K)
    out_ref[...] = jnp.concatenate(
        [acc[:, k * _L:(k + 1) * _L] for k in range(_G)], axis=0
    ).reshape(_G, 1, _L)


def kernel(Z):
    N, L, C = Z.shape
    z_all = Z.reshape(N * L, C)
    out = pl.pallas_call(
        _msm_kernel,
        grid=(N // _G,),
        in_specs=[
            pl.BlockSpec((N * L, C), lambda c: (0, 0)),
            pl.BlockSpec((_W, C), lambda c: (c, 0)),
        ],
        out_specs=pl.BlockSpec((_G, 1, L), lambda c: (c, 0, 0)),
        out_shape=jax.ShapeDtypeStruct((N, 1, L), jnp.float32),
        scratch_shapes=[pltpu.VMEM((N * L, 1), jnp.float32),
                        pltpu.VMEM((N * L, C), jnp.bfloat16)],
    )(z_all, z_all)
    return out.reshape(N, L)


# speedup vs baseline: 1.1458x; 1.0609x over previous
import jax
import jax.numpy as jnp
from jax.experimental import pallas as pl
from jax.experimental.pallas import tpu as pltpu

_N, _L, _C = 16, 256, 1024
_K = 4
_G = 4
_W = _G * _L
_NG = _N // _G

_NT = (((1,), (1,)), ((), ()))
_HI = jax.lax.Precision.HIGHEST


def _msm_kernel(z_all_ref, out_ref, zb_ref, nc_ref, nr_ref):
    c = pl.program_id(0)

    @pl.when(c == 0)
    def _():
        z_all = z_all_ref[...]
        n_all = jnp.sum(z_all * z_all, axis=1, keepdims=True)
        nc_ref[...] = n_all
        zb_ref[...] = z_all.astype(jnp.bfloat16)
        for k in range(_NG):
            nr_ref[k:k + 1, :] = jax.lax.dot_general(
                jnp.ones((1, 1), jnp.float32),
                n_all[k * _W:(k + 1) * _W, :], _NT,
                precision=_HI, preferred_element_type=jnp.float32)

    g = jax.lax.dot_general(zb_ref[...], zb_ref[pl.ds(c * _W, _W), :], _NT,
                            preferred_element_type=jnp.float32)
    h = nc_ref[...] - 2.0 * g
    n_p = nr_ref[pl.ds(c, 1), :]

    mins = [jnp.min(h[j * _L:(j + 1) * _L, :], axis=0, keepdims=True)
            for j in range(_N)]
    m = jnp.concatenate(mins, axis=0)

    row = jax.lax.broadcasted_iota(jnp.int32, (_N, _W), 0)
    img = c * _G + jax.lax.broadcasted_iota(jnp.int32, (_N, _W), 1) // _L
    inf = jnp.float32(jnp.inf)
    m = jnp.where(row == img, inf, m)

    acc = jnp.zeros((1, _W), jnp.float32)
    rem = jnp.full((1, _W), jnp.float32(_K))
    for _ in range(_K):
        v = jnp.min(m, axis=0, keepdims=True)
        cnt = jnp.sum((m == v).astype(jnp.float32), axis=0, keepdims=True)
        t = jnp.minimum(cnt, rem)
        d = jnp.sqrt(jnp.maximum(n_p + v, 1e-12))
        acc = acc + jnp.where(t > 0.0, t * d, 0.0)
        rem = rem - t
        m = jnp.where(m == v, inf, m)

    acc = acc / jnp.float32(_K)
    out_ref[...] = jnp.concatenate(
        [acc[:, k * _L:(k + 1) * _L] for k in range(_G)], axis=0
    ).reshape(_G, 1, _L)


def kernel(Z):
    N, L, C = Z.shape
    z_all = Z.reshape(N * L, C)
    out = pl.pallas_call(
        _msm_kernel,
        grid=(_NG,),
        in_specs=[pl.BlockSpec((N * L, C), lambda c: (0, 0))],
        out_specs=pl.BlockSpec((_G, 1, L), lambda c: (c, 0, 0)),
        out_shape=jax.ShapeDtypeStruct((N, 1, L), jnp.float32),
        scratch_shapes=[pltpu.VMEM((N * L, C), jnp.bfloat16),
                        pltpu.VMEM((N * L, 1), jnp.float32),
                        pltpu.VMEM((_NG, _W), jnp.float32)],
    )(z_all)
    return out.reshape(N, L)
